# baseline (speedup 1.0000x reference)
import functools

import jax
import jax.numpy as jnp
from jax.experimental import pallas as pl
from jax.experimental.pallas import tpu as pltpu

T = 2048
DIM = 1024
HID = 4096
E = 8
TOP_K = 2

BT = 512
HB = 512
NT = T // BT
NH = HID // HB

_INV_SQRT2 = 0.7071067811865476


def _gelu_exact(h):
    return 0.5 * h * (1.0 + jax.lax.erf(h * _INV_SQRT2))


def _dense_body(x_ref, gw_ref, gb_ref, w1_ref, b1_ref, w2_ref, b2_ref,
                out_ref, wsel_ref, xb_ref):
    e = pl.program_id(1)
    h = pl.program_id(2)

    @pl.when((e == 0) & (h == 0))
    def _gate():
        xf = x_ref[...]
        xb_ref[...] = xf.astype(jnp.bfloat16)
        logits = jax.lax.dot_general(
            xf, gw_ref[...], (((1,), (1,)), ((), ())),
            preferred_element_type=jnp.float32) + gb_ref[...]
        mx = jnp.max(logits, axis=-1, keepdims=True)
        ex = jnp.exp(logits - mx)
        p = ex / jnp.sum(ex, axis=-1, keepdims=True)
        iota8 = jax.lax.broadcasted_iota(jnp.int32, p.shape, 1)
        v1 = jnp.max(p, axis=-1, keepdims=True)
        i1 = jnp.min(jnp.where(p >= v1, iota8, E), axis=-1, keepdims=True)
        m1 = iota8 == i1
        p2 = jnp.where(m1, -jnp.inf, p)
        v2 = jnp.max(p2, axis=-1, keepdims=True)
        i2 = jnp.min(jnp.where(p2 >= v2, iota8, E), axis=-1, keepdims=True)
        m2 = iota8 == i2
        wsel_ref[...] = jnp.where(m1 | m2, p, 0.0)
        out_ref[...] = jnp.zeros_like(out_ref)

    iota8 = jax.lax.broadcasted_iota(jnp.int32, (BT, E), 1)
    scale = jnp.sum(jnp.where(iota8 == e, wsel_ref[...], 0.0),
                    axis=-1, keepdims=True)

    hm = jax.lax.dot_general(
        xb_ref[...], w1_ref[0], (((1,), (1,)), ((), ())),
        preferred_element_type=jnp.float32) + b1_ref[0, 0]
    g = _gelu_exact(hm).astype(jnp.bfloat16)
    part = jax.lax.dot_general(
        g, w2_ref[0], (((1,), (1,)), ((), ())),
        preferred_element_type=jnp.float32)

    @pl.when(h == 0)
    def _bias2():
        out_ref[...] += scale * b2_ref[0]

    out_ref[...] += scale * part


@jax.jit
def _moe_dense(x, gate_w, gate_b2d, w1b, b1, w2b, b2):
    return pl.pallas_call(
        _dense_body,
        grid=(NT, E, NH),
        in_specs=[
            pl.BlockSpec((BT, DIM), lambda t, e, h: (t, 0)),
            pl.BlockSpec((E, DIM), lambda t, e, h: (0, 0)),
            pl.BlockSpec((1, E), lambda t, e, h: (0, 0)),
            pl.BlockSpec((1, HB, DIM), lambda t, e, h: (e, h, 0)),
            pl.BlockSpec((1, 1, 1, HB), lambda t, e, h: (e, h, 0, 0)),
            pl.BlockSpec((1, DIM, HB), lambda t, e, h: (e, 0, h)),
            pl.BlockSpec((1, 1, DIM), lambda t, e, h: (e, 0, 0)),
        ],
        out_specs=pl.BlockSpec((BT, DIM), lambda t, e, h: (t, 0)),
        out_shape=jax.ShapeDtypeStruct((T, DIM), jnp.float32),
        scratch_shapes=[
            pltpu.VMEM((BT, E), jnp.float32),
            pltpu.VMEM((BT, DIM), jnp.bfloat16),
        ],
    )(x, gate_w, gate_b2d, w1b, b1, w2b, b2)


def kernel(x, gate_w, gate_b, w1, b1, w2, b2):
    w1b = w1.astype(jnp.bfloat16)
    w2b = w2.astype(jnp.bfloat16)
    return _moe_dense(x, gate_w, gate_b.reshape(1, E), w1b,
                      b1.reshape(E, NH, 1, HB), w2b, b2.reshape(E, 1, DIM))


# speedup vs baseline: 1.3393x; 1.3393x over previous
import functools

import jax
import jax.numpy as jnp
from jax import lax
from jax.experimental import pallas as pl
from jax.experimental.pallas import tpu as pltpu
from jax.experimental.pallas import tpu_sc as plsc

T = 2048
DIM = 1024
HID = 4096
E = 8
TOP_K = 2

BM = 256
NPAD = 6144
NB = NPAD // BM

NC = 2
NS = 16
NW = NC * NS
GCH = 64
_C_PER_W = T // NW
_C_CH = 32

_INV_SQRT2 = 0.7071067811865476


def _gate_body(x_ref, gw_ref, gb_ref, ev_ref, wv_ref):
    xf = x_ref[...]
    logits = lax.dot_general(
        xf, gw_ref[...], (((1,), (1,)), ((), ())),
        preferred_element_type=jnp.float32) + gb_ref[...]
    mx = jnp.max(logits, axis=-1, keepdims=True)
    ex = jnp.exp(logits - mx)
    p = ex / jnp.sum(ex, axis=-1, keepdims=True)
    iota8 = lax.broadcasted_iota(jnp.int32, p.shape, 1)
    v1 = jnp.max(p, axis=-1, keepdims=True)
    i1 = jnp.min(jnp.where(p >= v1, iota8, E), axis=-1, keepdims=True)
    m1 = iota8 == i1
    p2 = jnp.where(m1, -jnp.inf, p)
    v2 = jnp.max(p2, axis=-1, keepdims=True)
    i2 = jnp.min(jnp.where(p2 >= v2, iota8, E), axis=-1, keepdims=True)
    ev_ref[...] = jnp.concatenate([i1, i2], axis=1)
    wv_ref[...] = jnp.concatenate([v1, v2], axis=1)


def _gating(x, gate_w, gate_b2d):
    return pl.pallas_call(
        _gate_body,
        grid=(1,),
        in_specs=[
            pl.BlockSpec((T, DIM), lambda i: (0, 0)),
            pl.BlockSpec((E, DIM), lambda i: (0, 0)),
            pl.BlockSpec((1, E), lambda i: (0, 0)),
        ],
        out_specs=[
            pl.BlockSpec((T, TOP_K), lambda i: (0, 0)),
            pl.BlockSpec((T, TOP_K), lambda i: (0, 0)),
        ],
        out_shape=[
            jax.ShapeDtypeStruct((T, TOP_K), jnp.int32),
            jax.ShapeDtypeStruct((T, TOP_K), jnp.float32),
        ],
    )(x, gate_w, gate_b2d)


def _route_meta(ev, wv):
    fe = ev.reshape(-1)
    wf = wv.reshape(-1)
    oh = fe[:, None] == jnp.arange(E, dtype=jnp.int32)[None, :]
    csum = jnp.cumsum(oh.astype(jnp.int32), axis=0)
    counts = csum[-1]
    rank = jnp.sum(jnp.where(oh, csum - 1, 0), axis=1)
    padded = ((counts + BM - 1) // BM) * BM
    ends = jnp.cumsum(padded)
    offs = ends - padded
    dest = offs[fe] + rank
    row_token = jnp.zeros((NPAD,), jnp.int32).at[dest].set(
        jnp.arange(2 * T, dtype=jnp.int32) // TOP_K)
    row_weight = jnp.zeros((NPAD,), jnp.float32).at[dest].set(wf)
    bstart = jnp.arange(NB, dtype=jnp.int32) * BM
    block_expert = jnp.minimum(
        jnp.sum(bstart[:, None] >= ends[None, :], axis=1), E - 1
    ).astype(jnp.int32)
    nactive = (ends[-1] // BM).astype(jnp.int32)
    pe_plus = jnp.concatenate([block_expert, nactive[None]])
    pos0 = dest[0::2].astype(jnp.int32)
    pos1 = dest[1::2].astype(jnp.int32)
    return pe_plus, row_token, row_weight.reshape(NPAD, 1), pos0, pos1


_G_PER_W = NPAD // NW
_G_CHUNKS = _G_PER_W // GCH
_SC_BUILT = {}


def _build_sc_kernels():
    if "gather" in _SC_BUILT:
        return _SC_BUILT
    mesh = plsc.VectorSubcoreMesh(core_axis_name="c", subcore_axis_name="s")

    @functools.partial(
        pl.kernel, mesh=mesh,
        out_type=jax.ShapeDtypeStruct((NPAD, DIM), jnp.float32),
        scratch_types=[
            pltpu.VMEM((GCH,), jnp.int32),
            pltpu.VMEM((GCH, DIM), jnp.float32),
            pltpu.SemaphoreType.DMA,
        ],
    )
    def sc_gather(x_hbm, idx_hbm, out_hbm, idx_v, rows_v, sem):
        wid = lax.axis_index("s") * NC + lax.axis_index("c")
        base = wid * _G_PER_W
        for ci in range(_G_CHUNKS):
            off = base + ci * GCH
            pltpu.sync_copy(idx_hbm.at[pl.ds(off, GCH)], idx_v)
            pltpu.async_copy(x_hbm.at[idx_v], rows_v, sem).wait()
            pltpu.sync_copy(rows_v, out_hbm.at[pl.ds(off, GCH)])

    @functools.partial(
        pl.kernel, mesh=mesh,
        out_type=jax.ShapeDtypeStruct((T, DIM), jnp.float32),
        scratch_types=[
            pltpu.VMEM((_C_CH,), jnp.int32),
            pltpu.VMEM((_C_CH, DIM), jnp.float32),
            pltpu.VMEM((_C_CH, DIM), jnp.float32),
            pltpu.SemaphoreType.DMA,
        ],
    )
    def sc_combine(ys_hbm, pos0_hbm, pos1_hbm, out_hbm,
                   pos_v, rows_a, rows_b, sem):
        c = lax.axis_index("c")
        s = lax.axis_index("s")
        for k in range(_C_PER_W // _C_CH):
            tokbase = c * (NS * _C_PER_W) + s * _C_PER_W + k * _C_CH
            pltpu.sync_copy(pos0_hbm.at[pl.ds(tokbase, _C_CH)], pos_v)
            pltpu.async_copy(ys_hbm.at[pos_v], rows_a, sem).wait()
            pltpu.sync_copy(pos1_hbm.at[pl.ds(tokbase, _C_CH)], pos_v)
            pltpu.async_copy(ys_hbm.at[pos_v], rows_b, sem).wait()

            def addrow(r, _):
                for u in range(DIM // 16):
                    sl = pl.ds(u * 16, 16)
                    rows_a[r, sl] = rows_a[r, sl] + rows_b[r, sl]
                return 0

            lax.fori_loop(0, _C_CH, addrow, 0)
            pltpu.sync_copy(rows_a, out_hbm.at[pl.ds(tokbase, _C_CH)])

    _SC_BUILT["gather"] = sc_gather
    _SC_BUILT["combine"] = sc_combine
    return _SC_BUILT


def _sc_gather(x, row_token):
    return _build_sc_kernels()["gather"](x, row_token)


def _sc_combine(ys, pos0, pos1):
    return _build_sc_kernels()["combine"](ys, pos0, pos1)


def _gelu_exact(h):
    return 0.5 * h * (1.0 + lax.erf(h * _INV_SQRT2))


def _mlp_body(pe_ref, xs_ref, rw_ref, w1_ref, b1_ref, w2_ref, b2_ref, ys_ref):
    b = pl.program_id(0)

    @pl.when(b < pe_ref[NB])
    def _():
        xb = xs_ref[...].astype(jnp.bfloat16)
        h = lax.dot_general(
            xb, w1_ref[0], (((1,), (1,)), ((), ())),
            preferred_element_type=jnp.float32) + b1_ref[0]
        g = _gelu_exact(h).astype(jnp.bfloat16)
        y = lax.dot_general(
            g, w2_ref[0], (((1,), (1,)), ((), ())),
            preferred_element_type=jnp.float32)
        ys_ref[...] = (y + b2_ref[0]) * rw_ref[...]


def _mlp(pe_plus, xs, rw, w1b, b1r, w2b, b2r):
    grid_spec = pltpu.PrefetchScalarGridSpec(
        num_scalar_prefetch=1,
        grid=(NB,),
        in_specs=[
            pl.BlockSpec((BM, DIM), lambda b, pe: (b, 0)),
            pl.BlockSpec((BM, 1), lambda b, pe: (b, 0)),
            pl.BlockSpec((1, HID, DIM), lambda b, pe: (pe[b], 0, 0)),
            pl.BlockSpec((1, 1, HID), lambda b, pe: (pe[b], 0, 0)),
            pl.BlockSpec((1, DIM, HID), lambda b, pe: (pe[b], 0, 0)),
            pl.BlockSpec((1, 1, DIM), lambda b, pe: (pe[b], 0, 0)),
        ],
        out_specs=pl.BlockSpec((BM, DIM), lambda b, pe: (b, 0)),
    )
    return pl.pallas_call(
        _mlp_body,
        grid_spec=grid_spec,
        out_shape=jax.ShapeDtypeStruct((NPAD, DIM), jnp.float32),
    )(pe_plus, xs, rw, w1b, b1r, w2b, b2r)


@jax.jit
def _moe_dispatch(x, gate_w, gate_b2d, w1b, b1r, w2b, b2r):
    ev, wv = _gating(x, gate_w, gate_b2d)
    pe_plus, row_token, rw, pos0, pos1 = _route_meta(ev, wv)
    xs = _sc_gather(x, row_token)
    ys = _mlp(pe_plus, xs, rw, w1b, b1r, w2b, b2r)
    return _sc_combine(ys, pos0, pos1)


def kernel(x, gate_w, gate_b, w1, b1, w2, b2):
    w1b = w1.astype(jnp.bfloat16)
    w2b = w2.astype(jnp.bfloat16)
    return _moe_dispatch(x, gate_w, gate_b.reshape(1, E), w1b,
                         b1.reshape(E, 1, HID), w2b, b2.reshape(E, 1, DIM))
